# baseline (speedup 1.0000x reference)
import functools

import jax
import jax.numpy as jnp
from jax import lax
from jax.experimental import pallas as pl
from jax.experimental.pallas import tpu as pltpu
from jax.experimental.pallas import tpu_sc as plsc

NUM_CORES = 2
NUM_SUBCORES = 16
NW = NUM_CORES * NUM_SUBCORES
CH = 128


def _mesh():
    return plsc.VectorSubcoreMesh(
        core_axis_name="c", subcore_axis_name="s",
        num_cores=NUM_CORES, num_subcores=NUM_SUBCORES)


def _build_deg_kernel(npad, nch):
    rpt = npad // NUM_SUBCORES

    @functools.partial(
        pl.kernel,
        out_type=(
            jax.ShapeDtypeStruct((NUM_CORES, npad, 128), jnp.float32),
            jax.ShapeDtypeStruct((NUM_CORES, npad, 128), jnp.float32),
        ),
        mesh=_mesh(),
        scratch_types=[
            pltpu.VMEM_SHARED((npad, 128), jnp.float32),
            pltpu.VMEM((nch, CH), jnp.int32),
            pltpu.VMEM((CH, 128), jnp.float32),
        ],
    )
    def deg_kernel(src3, dst3, ones_hbm, zeros_hbm, degi_hbm, dego_hbm,
                   deg_sp, didx, ones_v):
        cid = lax.axis_index("c")
        sid = lax.axis_index("s")
        wid = sid * NUM_CORES + cid
        r0 = sid * rpt
        pltpu.sync_copy(ones_hbm.at[pl.ds(0, CH)], ones_v)

        def scatter_phase(idx3, deg_hbm):
            pltpu.sync_copy(zeros_hbm.at[pl.ds(r0, rpt)],
                            deg_sp.at[pl.ds(r0, rpt)])
            pltpu.sync_copy(idx3.at[wid], didx)
            plsc.subcore_barrier()

            def body(j, carry):
                pltpu.sync_copy(ones_v, deg_sp.at[didx.at[j]], add=True)
                return carry

            lax.fori_loop(0, nch, body, 0)
            plsc.subcore_barrier()
            pltpu.sync_copy(deg_sp.at[pl.ds(r0, rpt)],
                            deg_hbm.at[cid, pl.ds(r0, rpt)])
            plsc.subcore_barrier()

        scatter_phase(dst3, degi_hbm)
        scatter_phase(src3, dego_hbm)

    return deg_kernel


def _build_agg_kernel(npad, nch, width):
    rpt = npad // NUM_SUBCORES
    nchr = nch // 2

    @functools.partial(
        pl.kernel,
        out_type=jax.ShapeDtypeStruct((NUM_CORES, npad, width), jnp.float32),
        mesh=_mesh(),
        scratch_types=[
            pltpu.VMEM_SHARED((npad, width), jnp.float32),
            pltpu.VMEM((nchr, CH), jnp.int32),
            pltpu.VMEM((nchr, CH), jnp.int32),
            pltpu.VMEM((CH, width), jnp.float32),
            pltpu.VMEM((CH, width), jnp.float32),
            pltpu.SemaphoreType.DMA,
            pltpu.SemaphoreType.DMA,
        ],
    )
    def agg_kernel(hw_hbm, src3, dst3, zeros_hbm, out_hbm,
                   agg_sp, sidx, didx, rows0, rows1, sem0, sem1):
        cid = lax.axis_index("c")
        sid = lax.axis_index("s")
        wid = sid * NUM_CORES + cid
        r0 = sid * rpt
        pltpu.sync_copy(zeros_hbm.at[pl.ds(r0, rpt)], agg_sp.at[pl.ds(r0, rpt)])
        plsc.subcore_barrier()

        def round_body(r, carry):
            pltpu.sync_copy(src3.at[wid, pl.ds(r * nchr, nchr)], sidx)
            pltpu.sync_copy(dst3.at[wid, pl.ds(r * nchr, nchr)], didx)
            pltpu.async_copy(hw_hbm.at[sidx.at[0]], rows0, sem0)
            pltpu.async_copy(hw_hbm.at[sidx.at[1]], rows1, sem1)

            def body(i, carry2):
                j = 2 * i
                pltpu.make_async_copy(hw_hbm.at[sidx.at[j]], rows0, sem0).wait()
                pltpu.sync_copy(rows0, agg_sp.at[didx.at[j]], add=True)

                @pl.when(j + 2 < nchr)
                def _():
                    pltpu.async_copy(hw_hbm.at[sidx.at[j + 2]], rows0, sem0)

                pltpu.make_async_copy(hw_hbm.at[sidx.at[j + 1]], rows1, sem1).wait()
                pltpu.sync_copy(rows1, agg_sp.at[didx.at[j + 1]], add=True)

                @pl.when(j + 3 < nchr)
                def _():
                    pltpu.async_copy(hw_hbm.at[sidx.at[j + 3]], rows1, sem1)

                return carry2

            return lax.fori_loop(0, nchr // 2, body, carry)

        lax.fori_loop(0, 2, round_body, 0)
        plsc.subcore_barrier()
        pltpu.sync_copy(agg_sp.at[pl.ds(r0, rpt)], out_hbm.at[cid, pl.ds(r0, rpt)])

    return agg_kernel


def _deg_norm(degp_ref):
    deg = degp_ref[0, :, 0:1] + degp_ref[1, :, 0:1]
    return lax.rsqrt(jnp.maximum(deg, 1.0))


def _first_matmul(npad, bn):
    def body(x_ref, w_ref, dego_ref, degi_ref, o_ref, no_ref, ni_ref):
        normo = _deg_norm(dego_ref)
        normi = _deg_norm(degi_ref)
        no_ref[...] = jnp.broadcast_to(normo, (bn, 8))
        ni_ref[...] = jnp.broadcast_to(normi, (bn, 8))
        o_ref[...] = jnp.dot(x_ref[...], w_ref[...],
                             preferred_element_type=jnp.float32) * normo

    return pl.pallas_call(
        body,
        grid=(npad // bn,),
        in_specs=[
            pl.BlockSpec((bn, 128), lambda i: (i, 0)),
            pl.BlockSpec((128, 128), lambda i: (0, 0)),
            pl.BlockSpec((2, bn, 128), lambda i: (0, i, 0)),
            pl.BlockSpec((2, bn, 128), lambda i: (0, i, 0)),
        ],
        out_specs=[
            pl.BlockSpec((bn, 128), lambda i: (i, 0)),
            pl.BlockSpec((bn, 8), lambda i: (i, 0)),
            pl.BlockSpec((bn, 8), lambda i: (i, 0)),
        ],
        out_shape=[
            jax.ShapeDtypeStruct((npad, 128), jnp.float32),
            jax.ShapeDtypeStruct((npad, 8), jnp.float32),
            jax.ShapeDtypeStruct((npad, 8), jnp.float32),
        ],
    )


def _mid_matmul(npad, bn, outw):
    def body(p_ref, ni_ref, b_ref, w_ref, no_ref, o_ref):
        h = (p_ref[0] + p_ref[1]) * ni_ref[:, 0:1] + b_ref[...]
        h = jnp.maximum(h, 0.0)
        o_ref[...] = jnp.dot(h, w_ref[...],
                             preferred_element_type=jnp.float32) * no_ref[:, 0:1]

    return pl.pallas_call(
        body,
        grid=(npad // bn,),
        in_specs=[
            pl.BlockSpec((2, bn, 128), lambda i: (0, i, 0)),
            pl.BlockSpec((bn, 8), lambda i: (i, 0)),
            pl.BlockSpec((1, 128), lambda i: (0, 0)),
            pl.BlockSpec((128, outw), lambda i: (0, 0)),
            pl.BlockSpec((bn, 8), lambda i: (i, 0)),
        ],
        out_specs=pl.BlockSpec((bn, outw), lambda i: (i, 0)),
        out_shape=jax.ShapeDtypeStruct((npad, outw), jnp.float32),
    )


def _final_kernel(npad, bn, outw, nclass):
    def body(p_ref, ni_ref, b_ref, o_ref):
        z = (p_ref[0] + p_ref[1]) * ni_ref[:, 0:1] + b_ref[...]
        valid = lax.broadcasted_iota(jnp.int32, (1, outw), 1) < nclass
        zm = jnp.where(valid, z, -jnp.inf)
        m = jnp.max(zm, axis=1, keepdims=True)
        lse = m + jnp.log(jnp.sum(jnp.exp(zm - m), axis=1, keepdims=True))
        o_ref[...] = z - lse

    return pl.pallas_call(
        body,
        grid=(npad // bn,),
        in_specs=[
            pl.BlockSpec((2, bn, outw), lambda i: (0, i, 0)),
            pl.BlockSpec((bn, 8), lambda i: (i, 0)),
            pl.BlockSpec((1, outw), lambda i: (0, 0)),
        ],
        out_specs=pl.BlockSpec((bn, outw), lambda i: (i, 0)),
        out_shape=jax.ShapeDtypeStruct((npad, outw), jnp.float32),
    )


def kernel(x, edge_index, W1, b1, W2, b2, W3, b3):
    n, d = x.shape
    e = edge_index.shape[1]
    c = W3.shape[1]
    cpad = 128
    npad = (n // 512 + 1) * 512
    epw = -(-e // NW)
    ept = -(-epw // (4 * CH)) * (4 * CH)
    nch = ept // CH
    epad = ept * NW

    trash = n + (jnp.arange(epad - e, dtype=jnp.int32) % (npad - n))
    src3 = jnp.concatenate([edge_index[0], trash]).reshape(NW, nch, CH)
    dst3 = jnp.concatenate([edge_index[1], trash]).reshape(NW, nch, CH)

    xp = jnp.pad(x, ((0, npad - n), (0, 0)))
    w3p = jnp.pad(W3, ((0, 0), (0, cpad - c)))
    b1r = b1.reshape(1, -1)
    b2r = b2.reshape(1, -1)
    b3r = jnp.pad(b3, (0, cpad - c)).reshape(1, cpad)

    zeros128 = jnp.zeros((npad, 128), jnp.float32)
    ones128 = jnp.ones((npad, 128), jnp.float32)

    agg128 = _build_agg_kernel(npad, nch, 128)
    deg_k = _build_deg_kernel(npad, nch)
    bn = 512

    degi_p, dego_p = deg_k(src3, dst3, ones128, zeros128)

    hw1, normo, normi = _first_matmul(npad, bn)(xp, W1, dego_p, degi_p)
    p1 = agg128(hw1, src3, dst3, zeros128)
    hw2 = _mid_matmul(npad, bn, 128)(p1, normi, b1r, W2, normo)
    p2 = agg128(hw2, src3, dst3, zeros128)
    hw3 = _mid_matmul(npad, bn, cpad)(p2, normi, b2r, w3p, normo)
    p3 = agg128(hw3, src3, dst3, zeros128)
    out = _final_kernel(npad, bn, cpad, c)(p3, normi, b3r)

    return out[:n, :c]


# speedup vs baseline: 1.0018x; 1.0018x over previous
import functools

import jax
import jax.numpy as jnp
from jax import lax
from jax.experimental import pallas as pl
from jax.experimental.pallas import tpu as pltpu
from jax.experimental.pallas import tpu_sc as plsc

NUM_CORES = 2
NUM_SUBCORES = 16
NW = NUM_CORES * NUM_SUBCORES
CH = 128


def _mesh():
    return plsc.VectorSubcoreMesh(
        core_axis_name="c", subcore_axis_name="s",
        num_cores=NUM_CORES, num_subcores=NUM_SUBCORES)


def _build_deg_kernel(npad, nch):
    rpt = npad // NUM_SUBCORES

    @functools.partial(
        pl.kernel,
        out_type=jax.ShapeDtypeStruct((NUM_CORES, npad, 128), jnp.float32),
        mesh=_mesh(),
        scratch_types=[
            pltpu.VMEM_SHARED((npad, 128), jnp.float32),
            pltpu.VMEM((nch, CH), jnp.int32),
            pltpu.VMEM((CH, 128), jnp.float32),
        ],
    )
    def deg_kernel(idx3, ones_hbm, zeros_hbm, deg_hbm, deg_sp, didx, ones_v):
        cid = lax.axis_index("c")
        sid = lax.axis_index("s")
        wid = sid * NUM_CORES + cid
        r0 = sid * rpt
        pltpu.sync_copy(zeros_hbm.at[pl.ds(r0, rpt)], deg_sp.at[pl.ds(r0, rpt)])
        pltpu.sync_copy(ones_hbm.at[pl.ds(0, CH)], ones_v)
        pltpu.sync_copy(idx3.at[wid], didx)
        plsc.subcore_barrier()

        def body(j, carry):
            pltpu.sync_copy(ones_v, deg_sp.at[didx.at[j]], add=True)
            return carry

        lax.fori_loop(0, nch, body, 0)
        plsc.subcore_barrier()
        pltpu.sync_copy(deg_sp.at[pl.ds(r0, rpt)], deg_hbm.at[cid, pl.ds(r0, rpt)])

    return deg_kernel


def _build_agg_kernel(npad, nch, width):
    rpt = npad // NUM_SUBCORES
    nchr = nch // 2

    @functools.partial(
        pl.kernel,
        out_type=jax.ShapeDtypeStruct((NUM_CORES, npad, width), jnp.float32),
        mesh=_mesh(),
        scratch_types=[
            pltpu.VMEM_SHARED((npad, width), jnp.float32),
            pltpu.VMEM((nchr, CH), jnp.int32),
            pltpu.VMEM((nchr, CH), jnp.int32),
            pltpu.VMEM((CH, width), jnp.float32),
            pltpu.VMEM((CH, width), jnp.float32),
            pltpu.SemaphoreType.DMA,
            pltpu.SemaphoreType.DMA,
        ],
    )
    def agg_kernel(hw_hbm, src3, dst3, zeros_hbm, out_hbm,
                   agg_sp, sidx, didx, rows0, rows1, sem0, sem1):
        cid = lax.axis_index("c")
        sid = lax.axis_index("s")
        wid = sid * NUM_CORES + cid
        r0 = sid * rpt
        pltpu.sync_copy(zeros_hbm.at[pl.ds(r0, rpt)], agg_sp.at[pl.ds(r0, rpt)])
        plsc.subcore_barrier()

        def round_body(r, carry):
            pltpu.sync_copy(src3.at[wid, pl.ds(r * nchr, nchr)], sidx)
            pltpu.sync_copy(dst3.at[wid, pl.ds(r * nchr, nchr)], didx)
            pltpu.async_copy(hw_hbm.at[sidx.at[0]], rows0, sem0)
            pltpu.async_copy(hw_hbm.at[sidx.at[1]], rows1, sem1)

            def body(i, carry2):
                j = 2 * i
                pltpu.make_async_copy(hw_hbm.at[sidx.at[j]], rows0, sem0).wait()
                pltpu.sync_copy(rows0, agg_sp.at[didx.at[j]], add=True)

                @pl.when(j + 2 < nchr)
                def _():
                    pltpu.async_copy(hw_hbm.at[sidx.at[j + 2]], rows0, sem0)

                pltpu.make_async_copy(hw_hbm.at[sidx.at[j + 1]], rows1, sem1).wait()
                pltpu.sync_copy(rows1, agg_sp.at[didx.at[j + 1]], add=True)

                @pl.when(j + 3 < nchr)
                def _():
                    pltpu.async_copy(hw_hbm.at[sidx.at[j + 3]], rows1, sem1)

                return carry2

            return lax.fori_loop(0, nchr // 2, body, carry)

        lax.fori_loop(0, 2, round_body, 0)
        plsc.subcore_barrier()
        pltpu.sync_copy(agg_sp.at[pl.ds(r0, rpt)], out_hbm.at[cid, pl.ds(r0, rpt)])

    return agg_kernel


def _deg_norm(degp_ref):
    deg = degp_ref[0, :, 0:1] + degp_ref[1, :, 0:1]
    return lax.rsqrt(jnp.maximum(deg, 1.0))


def _first_matmul(npad, bn):
    def body(x_ref, w_ref, dego_ref, degi_ref, o_ref, no_ref, ni_ref):
        normo = _deg_norm(dego_ref)
        normi = _deg_norm(degi_ref)
        no_ref[...] = jnp.broadcast_to(normo, (bn, 8))
        ni_ref[...] = jnp.broadcast_to(normi, (bn, 8))
        o_ref[...] = jnp.dot(x_ref[...], w_ref[...],
                             preferred_element_type=jnp.float32) * normo

    return pl.pallas_call(
        body,
        grid=(npad // bn,),
        in_specs=[
            pl.BlockSpec((bn, 128), lambda i: (i, 0)),
            pl.BlockSpec((128, 128), lambda i: (0, 0)),
            pl.BlockSpec((2, bn, 128), lambda i: (0, i, 0)),
            pl.BlockSpec((2, bn, 128), lambda i: (0, i, 0)),
        ],
        out_specs=[
            pl.BlockSpec((bn, 128), lambda i: (i, 0)),
            pl.BlockSpec((bn, 8), lambda i: (i, 0)),
            pl.BlockSpec((bn, 8), lambda i: (i, 0)),
        ],
        out_shape=[
            jax.ShapeDtypeStruct((npad, 128), jnp.float32),
            jax.ShapeDtypeStruct((npad, 8), jnp.float32),
            jax.ShapeDtypeStruct((npad, 8), jnp.float32),
        ],
    )


def _mid_matmul(npad, bn, outw):
    def body(p_ref, ni_ref, b_ref, w_ref, no_ref, o_ref):
        h = (p_ref[0] + p_ref[1]) * ni_ref[:, 0:1] + b_ref[...]
        h = jnp.maximum(h, 0.0)
        o_ref[...] = jnp.dot(h, w_ref[...],
                             preferred_element_type=jnp.float32) * no_ref[:, 0:1]

    return pl.pallas_call(
        body,
        grid=(npad // bn,),
        in_specs=[
            pl.BlockSpec((2, bn, 128), lambda i: (0, i, 0)),
            pl.BlockSpec((bn, 8), lambda i: (i, 0)),
            pl.BlockSpec((1, 128), lambda i: (0, 0)),
            pl.BlockSpec((128, outw), lambda i: (0, 0)),
            pl.BlockSpec((bn, 8), lambda i: (i, 0)),
        ],
        out_specs=pl.BlockSpec((bn, outw), lambda i: (i, 0)),
        out_shape=jax.ShapeDtypeStruct((npad, outw), jnp.float32),
    )


def _final_kernel(npad, bn, outw, nclass):
    def body(p_ref, ni_ref, b_ref, o_ref):
        z = (p_ref[0] + p_ref[1]) * ni_ref[:, 0:1] + b_ref[...]
        valid = lax.broadcasted_iota(jnp.int32, (1, outw), 1) < nclass
        zm = jnp.where(valid, z, -jnp.inf)
        m = jnp.max(zm, axis=1, keepdims=True)
        lse = m + jnp.log(jnp.sum(jnp.exp(zm - m), axis=1, keepdims=True))
        o_ref[...] = z - lse

    return pl.pallas_call(
        body,
        grid=(npad // bn,),
        in_specs=[
            pl.BlockSpec((2, bn, outw), lambda i: (0, i, 0)),
            pl.BlockSpec((bn, 8), lambda i: (i, 0)),
            pl.BlockSpec((1, outw), lambda i: (0, 0)),
        ],
        out_specs=pl.BlockSpec((bn, outw), lambda i: (i, 0)),
        out_shape=jax.ShapeDtypeStruct((npad, outw), jnp.float32),
    )


def kernel(x, edge_index, W1, b1, W2, b2, W3, b3):
    n, d = x.shape
    e = edge_index.shape[1]
    c = W3.shape[1]
    cpad = 128
    npad = (n // 512 + 1) * 512
    epw = -(-e // NW)
    ept = -(-epw // (4 * CH)) * (4 * CH)
    nch = ept // CH
    epad = ept * NW

    trash = n + (jnp.arange(epad - e, dtype=jnp.int32) % (npad - n))
    src3 = jnp.concatenate([edge_index[0], trash]).reshape(NW, nch, CH)
    dst3 = jnp.concatenate([edge_index[1], trash]).reshape(NW, nch, CH)

    xp = jnp.pad(x, ((0, npad - n), (0, 0)))
    w3p = jnp.pad(W3, ((0, 0), (0, cpad - c)))
    b1r = b1.reshape(1, -1)
    b2r = b2.reshape(1, -1)
    b3r = jnp.pad(b3, (0, cpad - c)).reshape(1, cpad)

    zeros128 = jnp.zeros((npad, 128), jnp.float32)
    ones128 = jnp.ones((npad, 128), jnp.float32)

    agg128 = _build_agg_kernel(npad, nch, 128)
    deg_k = _build_deg_kernel(npad, nch)
    bn = 512

    degi_p = deg_k(dst3, ones128, zeros128)
    dego_p = deg_k(src3, ones128, zeros128)

    hw1, normo, normi = _first_matmul(npad, bn)(xp, W1, dego_p, degi_p)
    p1 = agg128(hw1, src3, dst3, zeros128)
    hw2 = _mid_matmul(npad, bn, 128)(p1, normi, b1r, W2, normo)
    p2 = agg128(hw2, src3, dst3, zeros128)
    hw3 = _mid_matmul(npad, bn, cpad)(p2, normi, b2r, w3p, normo)
    p3 = agg128(hw3, src3, dst3, zeros128)
    out = _final_kernel(npad, bn, cpad, c)(p3, normi, b3r)

    return out[:n, :c]


# speedup vs baseline: 1.0222x; 1.0204x over previous
import functools

import jax
import jax.numpy as jnp
from jax import lax
from jax.experimental import pallas as pl
from jax.experimental.pallas import tpu as pltpu
from jax.experimental.pallas import tpu_sc as plsc

NUM_CORES = 2
NUM_SUBCORES = 16
NW = NUM_CORES * NUM_SUBCORES
CH = 128


def _mesh():
    return plsc.VectorSubcoreMesh(
        core_axis_name="c", subcore_axis_name="s",
        num_cores=NUM_CORES, num_subcores=NUM_SUBCORES)


def _build_deg_kernel(npad, nch):
    rpt = npad // NUM_SUBCORES

    @functools.partial(
        pl.kernel,
        out_type=jax.ShapeDtypeStruct((NUM_CORES, npad, 128), jnp.float32),
        mesh=_mesh(),
        scratch_types=[
            pltpu.VMEM_SHARED((npad, 128), jnp.float32),
            pltpu.VMEM((nch, CH), jnp.int32),
            pltpu.VMEM((CH, 128), jnp.float32),
        ],
    )
    def deg_kernel(idx3, ones_hbm, zeros_hbm, deg_hbm, deg_sp, didx, ones_v):
        cid = lax.axis_index("c")
        sid = lax.axis_index("s")
        wid = sid * NUM_CORES + cid
        r0 = sid * rpt
        pltpu.sync_copy(zeros_hbm.at[pl.ds(r0, rpt)], deg_sp.at[pl.ds(r0, rpt)])
        pltpu.sync_copy(ones_hbm.at[pl.ds(0, CH)], ones_v)
        pltpu.sync_copy(idx3.at[wid], didx)
        plsc.subcore_barrier()

        def body(j, carry):
            pltpu.sync_copy(ones_v, deg_sp.at[didx.at[j]], add=True)
            return carry

        lax.fori_loop(0, nch, body, 0)
        plsc.subcore_barrier()
        pltpu.sync_copy(deg_sp.at[pl.ds(r0, rpt)], deg_hbm.at[cid, pl.ds(r0, rpt)])

    return deg_kernel


def _build_agg_kernel(npad, nch, width):
    rpt = npad // NUM_SUBCORES
    nchr = nch // 2

    @functools.partial(
        pl.kernel,
        out_type=jax.ShapeDtypeStruct((NUM_CORES, npad, width), jnp.float32),
        mesh=_mesh(),
        scratch_types=[
            pltpu.VMEM_SHARED((npad, width), jnp.float32),
            pltpu.VMEM((nchr, CH), jnp.int32),
            pltpu.VMEM((nchr, CH), jnp.int32),
            pltpu.VMEM((CH, width), jnp.float32),
            pltpu.VMEM((CH, width), jnp.float32),
            pltpu.SemaphoreType.DMA,
            pltpu.SemaphoreType.DMA,
        ],
    )
    def agg_kernel(hw_hbm, src3, dst3, zeros_hbm, out_hbm,
                   agg_sp, sidx, didx, rows0, rows1, sem0, sem1):
        cid = lax.axis_index("c")
        sid = lax.axis_index("s")
        wid = sid * NUM_CORES + cid
        r0 = sid * rpt
        pltpu.sync_copy(zeros_hbm.at[pl.ds(r0, rpt)], agg_sp.at[pl.ds(r0, rpt)])
        plsc.subcore_barrier()

        def round_body(r, carry):
            pltpu.sync_copy(src3.at[wid, pl.ds(r * nchr, nchr)], sidx)
            pltpu.sync_copy(dst3.at[wid, pl.ds(r * nchr, nchr)], didx)
            pltpu.async_copy(hw_hbm.at[sidx.at[0]], rows0, sem0)
            pltpu.async_copy(hw_hbm.at[sidx.at[1]], rows1, sem1)

            def body(i, carry2):
                j = 2 * i
                pltpu.make_async_copy(hw_hbm.at[sidx.at[j]], rows0, sem0).wait()
                pltpu.sync_copy(rows0, agg_sp.at[didx.at[j]], add=True)

                @pl.when(j + 2 < nchr)
                def _():
                    pltpu.async_copy(hw_hbm.at[sidx.at[j + 2]], rows0, sem0)

                pltpu.make_async_copy(hw_hbm.at[sidx.at[j + 1]], rows1, sem1).wait()
                pltpu.sync_copy(rows1, agg_sp.at[didx.at[j + 1]], add=True)

                @pl.when(j + 3 < nchr)
                def _():
                    pltpu.async_copy(hw_hbm.at[sidx.at[j + 3]], rows1, sem1)

                return carry2

            return lax.fori_loop(0, nchr // 2, body, carry)

        lax.fori_loop(0, 2, round_body, 0)
        plsc.subcore_barrier()
        pltpu.sync_copy(agg_sp.at[pl.ds(r0, rpt)], out_hbm.at[cid, pl.ds(r0, rpt)])

    return agg_kernel


def _deg_norm(degp_ref):
    deg = degp_ref[0, :, 0:1] + degp_ref[1, :, 0:1]
    return lax.rsqrt(jnp.maximum(deg, 1.0))


def _first_matmul(npad, bn):
    def body(x_ref, w_ref, dego_ref, o_ref):
        norm = _deg_norm(dego_ref)
        o_ref[...] = jnp.dot(x_ref[...], w_ref[...],
                             preferred_element_type=jnp.float32) * norm

    return pl.pallas_call(
        body,
        grid=(npad // bn,),
        in_specs=[
            pl.BlockSpec((bn, 128), lambda i: (i, 0)),
            pl.BlockSpec((128, 128), lambda i: (0, 0)),
            pl.BlockSpec((2, bn, 128), lambda i: (0, i, 0)),
        ],
        out_specs=pl.BlockSpec((bn, 128), lambda i: (i, 0)),
        out_shape=jax.ShapeDtypeStruct((npad, 128), jnp.float32),
    )


def _mid_matmul(npad, bn, outw):
    def body(p_ref, degi_ref, b_ref, w_ref, dego_ref, o_ref):
        h = (p_ref[0] + p_ref[1]) * _deg_norm(degi_ref) + b_ref[...]
        h = jnp.maximum(h, 0.0)
        o_ref[...] = jnp.dot(h, w_ref[...],
                             preferred_element_type=jnp.float32) * _deg_norm(dego_ref)

    return pl.pallas_call(
        body,
        grid=(npad // bn,),
        in_specs=[
            pl.BlockSpec((2, bn, 128), lambda i: (0, i, 0)),
            pl.BlockSpec((2, bn, 128), lambda i: (0, i, 0)),
            pl.BlockSpec((1, 128), lambda i: (0, 0)),
            pl.BlockSpec((128, outw), lambda i: (0, 0)),
            pl.BlockSpec((2, bn, 128), lambda i: (0, i, 0)),
        ],
        out_specs=pl.BlockSpec((bn, outw), lambda i: (i, 0)),
        out_shape=jax.ShapeDtypeStruct((npad, outw), jnp.float32),
    )


def _final_kernel(npad, bn, outw, nclass):
    def body(p_ref, degi_ref, b_ref, o_ref):
        z = (p_ref[0] + p_ref[1]) * _deg_norm(degi_ref) + b_ref[...]
        valid = lax.broadcasted_iota(jnp.int32, (1, outw), 1) < nclass
        zm = jnp.where(valid, z, -jnp.inf)
        m = jnp.max(zm, axis=1, keepdims=True)
        lse = m + jnp.log(jnp.sum(jnp.exp(zm - m), axis=1, keepdims=True))
        o_ref[...] = z - lse

    return pl.pallas_call(
        body,
        grid=(npad // bn,),
        in_specs=[
            pl.BlockSpec((2, bn, outw), lambda i: (0, i, 0)),
            pl.BlockSpec((2, bn, 128), lambda i: (0, i, 0)),
            pl.BlockSpec((1, outw), lambda i: (0, 0)),
        ],
        out_specs=pl.BlockSpec((bn, outw), lambda i: (i, 0)),
        out_shape=jax.ShapeDtypeStruct((npad, outw), jnp.float32),
    )


def kernel(x, edge_index, W1, b1, W2, b2, W3, b3):
    n, d = x.shape
    e = edge_index.shape[1]
    c = W3.shape[1]
    cpad = 128
    npad = (n // 512 + 1) * 512
    epw = -(-e // NW)
    ept = -(-epw // (4 * CH)) * (4 * CH)
    nch = ept // CH
    epad = ept * NW

    trash = n + (jnp.arange(epad - e, dtype=jnp.int32) % (npad - n))
    src3 = jnp.concatenate([edge_index[0], trash]).reshape(NW, nch, CH)
    dst3 = jnp.concatenate([edge_index[1], trash]).reshape(NW, nch, CH)

    xp = jnp.pad(x, ((0, npad - n), (0, 0)))
    w3p = jnp.pad(W3, ((0, 0), (0, cpad - c)))
    b1r = b1.reshape(1, -1)
    b2r = b2.reshape(1, -1)
    b3r = jnp.pad(b3, (0, cpad - c)).reshape(1, cpad)

    zeros128 = jnp.zeros((npad, 128), jnp.float32)
    ones128 = jnp.ones((npad, 128), jnp.float32)

    agg128 = _build_agg_kernel(npad, nch, 128)
    deg_k = _build_deg_kernel(npad, nch)
    bn = 512

    degi_p = deg_k(dst3, ones128, zeros128)
    dego_p = deg_k(src3, ones128, zeros128)

    hw1 = _first_matmul(npad, bn)(xp, W1, dego_p)
    p1 = agg128(hw1, src3, dst3, zeros128)
    hw2 = _mid_matmul(npad, bn, 128)(p1, degi_p, b1r, W2, dego_p)
    p2 = agg128(hw2, src3, dst3, zeros128)
    hw3 = _mid_matmul(npad, bn, cpad)(p2, degi_p, b2r, w3p, dego_p)
    p3 = agg128(hw3, src3, dst3, zeros128)
    out = _final_kernel(npad, bn, cpad, c)(p3, degi_p, b3r)

    return out[:n, :c]


# speedup vs baseline: 1.0490x; 1.0262x over previous
import functools

import jax
import jax.numpy as jnp
from jax import lax
from jax.experimental import pallas as pl
from jax.experimental.pallas import tpu as pltpu
from jax.experimental.pallas import tpu_sc as plsc

NUM_CORES = 2
NUM_SUBCORES = 16
NW = NUM_CORES * NUM_SUBCORES
CH = 128


def _mesh():
    return plsc.VectorSubcoreMesh(
        core_axis_name="c", subcore_axis_name="s",
        num_cores=NUM_CORES, num_subcores=NUM_SUBCORES)


def _build_deg_kernel(npad, nch):
    rpt = npad // NUM_SUBCORES

    @functools.partial(
        pl.kernel,
        out_type=jax.ShapeDtypeStruct((NUM_CORES, npad, 128), jnp.float32),
        mesh=_mesh(),
        scratch_types=[
            pltpu.VMEM_SHARED((npad, 128), jnp.float32),
            pltpu.VMEM((nch, CH), jnp.int32),
            pltpu.VMEM((CH, 128), jnp.float32),
        ],
    )
    def deg_kernel(idx3, ones_hbm, zeros_hbm, deg_hbm, deg_sp, didx, ones_v):
        cid = lax.axis_index("c")
        sid = lax.axis_index("s")
        wid = sid * NUM_CORES + cid
        r0 = sid * rpt
        pltpu.sync_copy(zeros_hbm.at[pl.ds(r0, rpt)], deg_sp.at[pl.ds(r0, rpt)])
        pltpu.sync_copy(ones_hbm.at[pl.ds(0, CH)], ones_v)
        pltpu.sync_copy(idx3.at[wid], didx)
        plsc.subcore_barrier()

        def body(j, carry):
            pltpu.sync_copy(ones_v, deg_sp.at[didx.at[j]], add=True)
            return carry

        lax.fori_loop(0, nch, body, 0)
        plsc.subcore_barrier()
        pltpu.sync_copy(deg_sp.at[pl.ds(r0, rpt)], deg_hbm.at[cid, pl.ds(r0, rpt)])

    return deg_kernel


def _build_agg_kernel(npad, nch, width):
    rpt = npad // NUM_SUBCORES
    nchr = nch // 2

    @functools.partial(
        pl.kernel,
        out_type=jax.ShapeDtypeStruct((NUM_CORES, npad, width), jnp.float32),
        mesh=_mesh(),
        scratch_types=[
            pltpu.VMEM_SHARED((npad, width), jnp.float32),
            pltpu.VMEM((nchr, CH), jnp.int32),
            pltpu.VMEM((nchr, CH), jnp.int32),
            pltpu.VMEM((CH, width), jnp.float32),
            pltpu.VMEM((CH, width), jnp.float32),
            pltpu.SemaphoreType.DMA,
            pltpu.SemaphoreType.DMA,
        ],
    )
    def agg_kernel(hw_hbm, src3, dst3, zeros_hbm, out_hbm,
                   agg_sp, sidx, didx, rows0, rows1, sem0, sem1):
        cid = lax.axis_index("c")
        sid = lax.axis_index("s")
        wid = sid * NUM_CORES + cid
        r0 = sid * rpt
        pltpu.sync_copy(zeros_hbm.at[pl.ds(r0, rpt)], agg_sp.at[pl.ds(r0, rpt)])
        plsc.subcore_barrier()

        def round_body(r, carry):
            pltpu.sync_copy(src3.at[wid, pl.ds(r * nchr, nchr)], sidx)
            pltpu.sync_copy(dst3.at[wid, pl.ds(r * nchr, nchr)], didx)
            pltpu.async_copy(hw_hbm.at[sidx.at[0]], rows0, sem0)
            pltpu.async_copy(hw_hbm.at[sidx.at[1]], rows1, sem1)

            def body(i, carry2):
                j = 2 * i
                pltpu.make_async_copy(hw_hbm.at[sidx.at[j]], rows0, sem0).wait()
                pltpu.sync_copy(rows0, agg_sp.at[didx.at[j]], add=True)

                @pl.when(j + 2 < nchr)
                def _():
                    pltpu.async_copy(hw_hbm.at[sidx.at[j + 2]], rows0, sem0)

                pltpu.make_async_copy(hw_hbm.at[sidx.at[j + 1]], rows1, sem1).wait()
                pltpu.sync_copy(rows1, agg_sp.at[didx.at[j + 1]], add=True)

                @pl.when(j + 3 < nchr)
                def _():
                    pltpu.async_copy(hw_hbm.at[sidx.at[j + 3]], rows1, sem1)

                return carry2

            return lax.fori_loop(0, nchr // 2, body, carry)

        lax.fori_loop(0, 2, round_body, 0)
        plsc.subcore_barrier()
        pltpu.sync_copy(agg_sp.at[pl.ds(r0, rpt)], out_hbm.at[cid, pl.ds(r0, rpt)])

    return agg_kernel


def _deg_norm(degp_ref):
    deg = degp_ref[0, :, 0:1] + degp_ref[1, :, 0:1]
    return lax.rsqrt(jnp.maximum(deg, 1.0))


def _first_matmul(npad, bn):
    def body(x_ref, w_ref, dego_ref, o_ref):
        norm = _deg_norm(dego_ref)
        o_ref[...] = jnp.dot(x_ref[...], w_ref[...],
                             preferred_element_type=jnp.float32) * norm

    return pl.pallas_call(
        body,
        grid=(npad // bn,),
        in_specs=[
            pl.BlockSpec((bn, 128), lambda i: (i, 0)),
            pl.BlockSpec((128, 128), lambda i: (0, 0)),
            pl.BlockSpec((2, bn, 128), lambda i: (0, i, 0)),
        ],
        out_specs=pl.BlockSpec((bn, 128), lambda i: (i, 0)),
        out_shape=jax.ShapeDtypeStruct((npad, 128), jnp.float32),
    )


def _mid_matmul(npad, bn, outw):
    def body(p_ref, degi_ref, b_ref, w_ref, dego_ref, o_ref):
        h = (p_ref[0] + p_ref[1]) * _deg_norm(degi_ref) + b_ref[...]
        h = jnp.maximum(h, 0.0)
        o_ref[...] = jnp.dot(h, w_ref[...],
                             preferred_element_type=jnp.float32) * _deg_norm(dego_ref)

    return pl.pallas_call(
        body,
        grid=(npad // bn,),
        in_specs=[
            pl.BlockSpec((2, bn, 128), lambda i: (0, i, 0)),
            pl.BlockSpec((2, bn, 128), lambda i: (0, i, 0)),
            pl.BlockSpec((1, 128), lambda i: (0, 0)),
            pl.BlockSpec((128, outw), lambda i: (0, 0)),
            pl.BlockSpec((2, bn, 128), lambda i: (0, i, 0)),
        ],
        out_specs=pl.BlockSpec((bn, outw), lambda i: (i, 0)),
        out_shape=jax.ShapeDtypeStruct((npad, outw), jnp.float32),
    )


def _final_kernel(npad, bn, outw, nclass):
    def body(p_ref, degi_ref, b_ref, o_ref):
        z = (p_ref[0] + p_ref[1]) * _deg_norm(degi_ref) + b_ref[...]
        valid = lax.broadcasted_iota(jnp.int32, (1, outw), 1) < nclass
        zm = jnp.where(valid, z, -jnp.inf)
        m = jnp.max(zm, axis=1, keepdims=True)
        lse = m + jnp.log(jnp.sum(jnp.exp(zm - m), axis=1, keepdims=True))
        o_ref[...] = z - lse

    return pl.pallas_call(
        body,
        grid=(npad // bn,),
        in_specs=[
            pl.BlockSpec((2, bn, outw), lambda i: (0, i, 0)),
            pl.BlockSpec((2, bn, 128), lambda i: (0, i, 0)),
            pl.BlockSpec((1, outw), lambda i: (0, 0)),
        ],
        out_specs=pl.BlockSpec((bn, outw), lambda i: (i, 0)),
        out_shape=jax.ShapeDtypeStruct((npad, outw), jnp.float32),
    )


def kernel(x, edge_index, W1, b1, W2, b2, W3, b3):
    n, d = x.shape
    e = edge_index.shape[1]
    c = W3.shape[1]
    cpad = 128
    npad = (n // 512 + 1) * 512
    epw = -(-e // NW)
    ept = -(-epw // (4 * CH)) * (4 * CH)
    nch = ept // CH
    epad = ept * NW

    trash = n + (jnp.arange(epad - e, dtype=jnp.int32) % (npad - n))
    src3 = jnp.concatenate([edge_index[0], trash]).reshape(NW, nch, CH)
    dst3 = jnp.concatenate([edge_index[1], trash]).reshape(NW, nch, CH)

    xp = jnp.pad(x, ((0, npad - n), (0, 0)))
    w3p = jnp.pad(W3, ((0, 0), (0, cpad - c)))
    b1r = b1.reshape(1, -1)
    b2r = b2.reshape(1, -1)
    b3r = jnp.pad(b3, (0, cpad - c)).reshape(1, cpad)

    zeros128 = jnp.zeros((npad, 128), jnp.float32)
    ones128 = jnp.ones((npad, 128), jnp.float32)

    agg128 = _build_agg_kernel(npad, nch, 128)
    deg_k = _build_deg_kernel(npad, nch)
    bn = 1024

    degi_p = deg_k(dst3, ones128, zeros128)
    dego_p = deg_k(src3, ones128, zeros128)

    hw1 = _first_matmul(npad, bn)(xp, W1, dego_p)
    p1 = agg128(hw1, src3, dst3, zeros128)
    hw2 = _mid_matmul(npad, bn, 128)(p1, degi_p, b1r, W2, dego_p)
    p2 = agg128(hw2, src3, dst3, zeros128)
    hw3 = _mid_matmul(npad, bn, cpad)(p2, degi_p, b2r, w3p, dego_p)
    p3 = agg128(hw3, src3, dst3, zeros128)
    out = _final_kernel(npad, bn, cpad, c)(p3, degi_p, b3r)

    return out[:n, :c]


# speedup vs baseline: 1.0629x; 1.0132x over previous
import functools

import jax
import jax.numpy as jnp
from jax import lax
from jax.experimental import pallas as pl
from jax.experimental.pallas import tpu as pltpu
from jax.experimental.pallas import tpu_sc as plsc

NUM_CORES = 2
NUM_SUBCORES = 16
NW = NUM_CORES * NUM_SUBCORES
CH = 128


def _mesh():
    return plsc.VectorSubcoreMesh(
        core_axis_name="c", subcore_axis_name="s",
        num_cores=NUM_CORES, num_subcores=NUM_SUBCORES)


def _build_deg_kernel(npad, nch):
    rpt = npad // NUM_SUBCORES

    @functools.partial(
        pl.kernel,
        out_type=jax.ShapeDtypeStruct((NUM_CORES, npad, 128), jnp.float32),
        mesh=_mesh(),
        scratch_types=[
            pltpu.VMEM_SHARED((npad, 128), jnp.float32),
            pltpu.VMEM((nch, CH), jnp.int32),
            pltpu.VMEM((CH, 128), jnp.float32),
        ],
    )
    def deg_kernel(idx3, ones_hbm, zeros_hbm, deg_hbm, deg_sp, didx, ones_v):
        cid = lax.axis_index("c")
        sid = lax.axis_index("s")
        wid = sid * NUM_CORES + cid
        r0 = sid * rpt
        pltpu.sync_copy(zeros_hbm.at[pl.ds(r0, rpt)], deg_sp.at[pl.ds(r0, rpt)])
        pltpu.sync_copy(ones_hbm.at[pl.ds(0, CH)], ones_v)
        pltpu.sync_copy(idx3.at[wid], didx)
        plsc.subcore_barrier()

        def body(j, carry):
            pltpu.sync_copy(ones_v, deg_sp.at[didx.at[j]], add=True)
            return carry

        lax.fori_loop(0, nch, body, 0)
        plsc.subcore_barrier()
        pltpu.sync_copy(deg_sp.at[pl.ds(r0, rpt)], deg_hbm.at[cid, pl.ds(r0, rpt)])

    return deg_kernel


def _build_agg_kernel(npad, nch, width):
    rpt = npad // NUM_SUBCORES
    nchr = nch // 2

    @functools.partial(
        pl.kernel,
        out_type=jax.ShapeDtypeStruct((NUM_CORES, npad, width), jnp.float32),
        mesh=_mesh(),
        scratch_types=[
            pltpu.VMEM_SHARED((npad, width), jnp.float32),
            pltpu.VMEM((nchr, CH), jnp.int32),
            pltpu.VMEM((nchr, CH), jnp.int32),
            pltpu.VMEM((CH, width), jnp.float32),
            pltpu.VMEM((CH, width), jnp.float32),
            pltpu.SemaphoreType.DMA,
            pltpu.SemaphoreType.DMA,
        ],
    )
    def agg_kernel(hw_hbm, src3, dst3, zeros_hbm, out_hbm,
                   agg_sp, sidx, didx, rows0, rows1, sem0, sem1):
        cid = lax.axis_index("c")
        sid = lax.axis_index("s")
        wid = sid * NUM_CORES + cid
        r0 = sid * rpt
        pltpu.sync_copy(zeros_hbm.at[pl.ds(r0, rpt)], agg_sp.at[pl.ds(r0, rpt)])
        plsc.subcore_barrier()

        def round_body(r, carry):
            pltpu.sync_copy(src3.at[wid, pl.ds(r * nchr, nchr)], sidx)
            pltpu.sync_copy(dst3.at[wid, pl.ds(r * nchr, nchr)], didx)
            pltpu.async_copy(hw_hbm.at[sidx.at[0]], rows0, sem0)
            pltpu.async_copy(hw_hbm.at[sidx.at[1]], rows1, sem1)

            def body(i, carry2):
                j = 2 * i
                pltpu.make_async_copy(hw_hbm.at[sidx.at[j]], rows0, sem0).wait()
                pltpu.sync_copy(rows0, agg_sp.at[didx.at[j]], add=True)

                @pl.when(j + 2 < nchr)
                def _():
                    pltpu.async_copy(hw_hbm.at[sidx.at[j + 2]], rows0, sem0)

                pltpu.make_async_copy(hw_hbm.at[sidx.at[j + 1]], rows1, sem1).wait()
                pltpu.sync_copy(rows1, agg_sp.at[didx.at[j + 1]], add=True)

                @pl.when(j + 3 < nchr)
                def _():
                    pltpu.async_copy(hw_hbm.at[sidx.at[j + 3]], rows1, sem1)

                return carry2

            return lax.fori_loop(0, nchr // 2, body, carry)

        lax.fori_loop(0, 2, round_body, 0)
        plsc.subcore_barrier()
        pltpu.sync_copy(agg_sp.at[pl.ds(r0, rpt)], out_hbm.at[cid, pl.ds(r0, rpt)])

    return agg_kernel


def _deg_norm(degp_ref):
    deg = degp_ref[0, :, 0:1] + degp_ref[1, :, 0:1]
    return lax.rsqrt(jnp.maximum(deg, 1.0))


def _first_matmul(npad, bn):
    def body(x_ref, w_ref, dego_ref, o_ref):
        norm = _deg_norm(dego_ref)
        o_ref[...] = jnp.dot(x_ref[...], w_ref[...],
                             preferred_element_type=jnp.float32) * norm

    return pl.pallas_call(
        body,
        grid=(npad // bn,),
        in_specs=[
            pl.BlockSpec((bn, 128), lambda i: (i, 0)),
            pl.BlockSpec((128, 128), lambda i: (0, 0)),
            pl.BlockSpec((2, bn, 128), lambda i: (0, i, 0)),
        ],
        out_specs=pl.BlockSpec((bn, 128), lambda i: (i, 0)),
        out_shape=jax.ShapeDtypeStruct((npad, 128), jnp.float32),
    )


def _mid_matmul(npad, bn, outw):
    def body(p_ref, degi_ref, b_ref, w_ref, dego_ref, o_ref):
        h = (p_ref[0] + p_ref[1]) * _deg_norm(degi_ref) + b_ref[...]
        h = jnp.maximum(h, 0.0)
        o_ref[...] = jnp.dot(h, w_ref[...],
                             preferred_element_type=jnp.float32) * _deg_norm(dego_ref)

    return pl.pallas_call(
        body,
        grid=(npad // bn,),
        in_specs=[
            pl.BlockSpec((2, bn, 128), lambda i: (0, i, 0)),
            pl.BlockSpec((2, bn, 128), lambda i: (0, i, 0)),
            pl.BlockSpec((1, 128), lambda i: (0, 0)),
            pl.BlockSpec((128, outw), lambda i: (0, 0)),
            pl.BlockSpec((2, bn, 128), lambda i: (0, i, 0)),
        ],
        out_specs=pl.BlockSpec((bn, outw), lambda i: (i, 0)),
        out_shape=jax.ShapeDtypeStruct((npad, outw), jnp.float32),
    )


def _final_kernel(npad, bn, outw, nclass):
    def body(p_ref, degi_ref, b_ref, o_ref):
        z = (p_ref[0] + p_ref[1]) * _deg_norm(degi_ref) + b_ref[...]
        valid = lax.broadcasted_iota(jnp.int32, (1, outw), 1) < nclass
        zm = jnp.where(valid, z, -jnp.inf)
        m = jnp.max(zm, axis=1, keepdims=True)
        lse = m + jnp.log(jnp.sum(jnp.exp(zm - m), axis=1, keepdims=True))
        o_ref[...] = z - lse

    return pl.pallas_call(
        body,
        grid=(npad // bn,),
        in_specs=[
            pl.BlockSpec((2, bn, outw), lambda i: (0, i, 0)),
            pl.BlockSpec((2, bn, 128), lambda i: (0, i, 0)),
            pl.BlockSpec((1, outw), lambda i: (0, 0)),
        ],
        out_specs=pl.BlockSpec((bn, outw), lambda i: (i, 0)),
        out_shape=jax.ShapeDtypeStruct((npad, outw), jnp.float32),
    )


def kernel(x, edge_index, W1, b1, W2, b2, W3, b3):
    n, d = x.shape
    e = edge_index.shape[1]
    c = W3.shape[1]
    cpad = 128
    npad = (n // 512 + 1) * 512
    epw = -(-e // NW)
    ept = -(-epw // (4 * CH)) * (4 * CH)
    nch = ept // CH
    epad = ept * NW

    trash = n + (jnp.arange(epad - e, dtype=jnp.int32) % (npad - n))
    src3 = jnp.concatenate([edge_index[0], trash]).reshape(NW, nch, CH)
    dst3 = jnp.concatenate([edge_index[1], trash]).reshape(NW, nch, CH)

    xp = jnp.pad(x, ((0, npad - n), (0, 0)))
    w3p = jnp.pad(W3, ((0, 0), (0, cpad - c)))
    b1r = b1.reshape(1, -1)
    b2r = b2.reshape(1, -1)
    b3r = jnp.pad(b3, (0, cpad - c)).reshape(1, cpad)

    zeros128 = jnp.zeros((npad, 128), jnp.float32)
    ones128 = jnp.ones((npad, 128), jnp.float32)

    agg128 = _build_agg_kernel(npad, nch, 128)
    deg_k = _build_deg_kernel(npad, nch)
    bn = 2048

    degi_p = deg_k(dst3, ones128, zeros128)
    dego_p = deg_k(src3, ones128, zeros128)

    hw1 = _first_matmul(npad, bn)(xp, W1, dego_p)
    p1 = agg128(hw1, src3, dst3, zeros128)
    hw2 = _mid_matmul(npad, bn, 128)(p1, degi_p, b1r, W2, dego_p)
    p2 = agg128(hw2, src3, dst3, zeros128)
    hw3 = _mid_matmul(npad, bn, cpad)(p2, degi_p, b2r, w3p, dego_p)
    p3 = agg128(hw3, src3, dst3, zeros128)
    out = _final_kernel(npad, bn, cpad, c)(p3, degi_p, b3r)

    return out[:n, :c]
